# Initial kernel scaffold; baseline (speedup 1.0000x reference)
import jax
import jax.numpy as jnp
from jax.experimental import pallas as pl


def kernel(x, router, w_up_gate, w_down):
    raise NotImplementedError("write your pallas kernel here")


# speedup vs baseline: 1.9104x; 1.9104x over previous
import functools

import jax
import jax.numpy as jnp
from jax.experimental import pallas as pl
from jax.experimental.pallas import tpu as pltpu

K = 2


def _moe_dense_body(x_ref, r_ref, wug_ref, wd_ref,
                    out_ref, counts_ref, ent_ref):
    i = pl.program_id(0)
    nsteps = pl.num_programs(0)
    xt = x_ref[...]
    tile, d = xt.shape
    e_num = r_ref.shape[1]

    lg = jax.lax.dot(xt, r_ref[...])

    iota_e = jax.lax.broadcasted_iota(jnp.int32, (tile, e_num), 1)
    l0 = jnp.max(lg, axis=-1, keepdims=True)
    e0 = jnp.min(jnp.where(lg == l0, iota_e, e_num), axis=-1)
    lg1 = jnp.where(iota_e == e0[:, None], -jnp.inf, lg)
    l1 = jnp.max(lg1, axis=-1, keepdims=True)
    e1 = jnp.min(jnp.where(lg1 == l1, iota_e, e_num), axis=-1)

    z = jnp.exp(l1[:, 0] - l0[:, 0])
    w0 = 1.0 / (1.0 + z)
    w1 = 1.0 - w0

    onehot = ((e0[:, None] == iota_e).astype(jnp.float32)
              + (e1[:, None] == iota_e).astype(jnp.float32))
    tile_counts = jnp.sum(onehot, axis=0)

    @pl.when(i == 0)
    def _():
        counts_ref[...] = jnp.zeros_like(counts_ref)
    counts_ref[...] += tile_counts[None, :]

    acc = jnp.zeros((tile, d), dtype=jnp.float32)
    for e in range(e_num):
        w_tok = (jnp.where(e0 == e, w0, 0.0) + jnp.where(e1 == e, w1, 0.0))
        ug = jax.lax.dot(xt, wug_ref[e])
        up, gate = jnp.split(ug, 2, axis=-1)
        h = up * (gate / (1.0 + jnp.exp(-gate)))
        y = jax.lax.dot(h, wd_ref[e])
        acc = acc + w_tok[:, None] * y
    out_ref[...] = acc

    @pl.when(i == nsteps - 1)
    def _():
        counts = counts_ref[0, :]
        total = jnp.maximum(jnp.sum(counts), 1.0)
        loads = counts / total
        ent_ref[...] = (-jnp.sum(loads * jnp.log(loads + 1e-6))).reshape(1, 1)


def kernel(x, router, w_up_gate, w_down):
    b, s, d = x.shape
    e_num = router.shape[1]
    t = b * s
    x_flat = x.reshape(t, d)
    tile = 256 if t % 256 == 0 else t

    grid = (t // tile,)
    out, counts, ent = pl.pallas_call(
        _moe_dense_body,
        grid=grid,
        in_specs=[
            pl.BlockSpec((tile, d), lambda i: (i, 0)),
            pl.BlockSpec(router.shape, lambda i: (0, 0)),
            pl.BlockSpec(w_up_gate.shape, lambda i: (0, 0, 0)),
            pl.BlockSpec(w_down.shape, lambda i: (0, 0, 0)),
        ],
        out_specs=[
            pl.BlockSpec((tile, d), lambda i: (i, 0)),
            pl.BlockSpec((1, e_num), lambda i: (0, 0)),
            pl.BlockSpec((1, 1), lambda i: (0, 0)),
        ],
        out_shape=[
            jax.ShapeDtypeStruct((t, d), jnp.float32),
            jax.ShapeDtypeStruct((1, e_num), jnp.float32),
            jax.ShapeDtypeStruct((1, 1), jnp.float32),
        ],
        compiler_params=pltpu.CompilerParams(
            vmem_limit_bytes=100 * 1024 * 1024),
    )(x_flat, router, w_up_gate, w_down)
    return out.reshape(b, s, d), counts[0], ent[0, 0]
